# baseline (speedup 1.0000x reference)
import functools

import jax
import jax.numpy as jnp
from jax import lax
from jax.experimental import pallas as pl
from jax.experimental.pallas import tpu as pltpu
from jax.experimental.pallas import tpu_sc as plsc

N_FIELDS = 26
VOCAB = 100000
DIM = 32
BATCH = 16384

VPAD = 100352
NVB = VPAD // 512
PACK_R = N_FIELDS * NVB * 128
G = 28

NC, NS = 2, 16
NW = NC * NS
BW = BATCH // NW
GCHUNK = 128
NG = BW // GCHUNK
NV = BW // 16


def _tc_relayout(tbl_m, wrow):
    def body(x_ref, w_ref, o_ref):
        xs = (x_ref[...] * w_ref[0]).astype(jnp.bfloat16)
        for g in range(G):
            o_ref[pl.ds(g * 128, 128), :] = jnp.concatenate(
                [xs[:, (g * 4 + q) * 128:(g * 4 + q + 1) * 128].T
                 for q in range(4)],
                axis=1,
            )

    return pl.pallas_call(
        body,
        out_shape=jax.ShapeDtypeStruct((PACK_R, 128), jnp.bfloat16),
        grid=(N_FIELDS, NVB // G),
        in_specs=[
            pl.BlockSpec((DIM, 512 * G), lambda f, v: (f, v)),
            pl.BlockSpec((1, 1, 512 * G), lambda f, v: (f, 0, 0)),
        ],
        out_specs=pl.BlockSpec(
            (128 * G, 128), lambda f, v: (f * (NVB // G) + v, 0)
        ),
    )(tbl_m, wrow)


def kernel(indices, tables, weights):
    tbl_m = tables.transpose(0, 2, 1).reshape(N_FIELDS * DIM, VOCAB + 1)
    wrow = jnp.broadcast_to(weights[:, None, None], (N_FIELDS, 1, 512 * G))
    flat = _tc_relayout(tbl_m, wrow).reshape(PACK_R * 4, DIM)
    idx_t = indices.T
    mesh = plsc.VectorSubcoreMesh(core_axis_name="c", subcore_axis_name="s")

    @functools.partial(
        pl.kernel,
        out_type=jax.ShapeDtypeStruct((BATCH, N_FIELDS * DIM), jnp.bfloat16),
        mesh=mesh,
        compiler_params=pltpu.CompilerParams(use_tc_tiling_on_sc=False),
        scratch_types=[
            pltpu.VMEM((BW,), jnp.int32),
            pltpu.VMEM((BW, DIM), jnp.bfloat16),
            pltpu.SemaphoreType.DMA,
        ],
    )
    def fk(idx_hbm, tbl_hbm, out_hbm, g_v, rows_v, sem):
        wid = lax.axis_index("s") * NC + lax.axis_index("c")
        b0 = pl.multiple_of(wid * BW, BW)

        def field_body(f, carry):
            pltpu.sync_copy(idx_hbm.at[f, pl.ds(b0, BW)], g_v)
            for k in range(NV):
                raw = g_v[pl.ds(k * 16, 16)]
                ok = (raw >= 0) & (raw < VOCAB)
                vs = jnp.where(ok, raw + 1, 0)
                row = (
                    f * VPAD
                    + (vs >> 9) * 512
                    + (vs & 127) * 4
                    + ((vs >> 7) & 3)
                )
                g_v[pl.ds(k * 16, 16)] = row
            for j in range(NG):
                pltpu.async_copy(
                    tbl_hbm.at[g_v.at[pl.ds(j * GCHUNK, GCHUNK)]],
                    rows_v.at[pl.ds(j * GCHUNK, GCHUNK)],
                    sem,
                )
            for j in range(NG):
                pltpu.make_async_copy(
                    tbl_hbm.at[g_v.at[pl.ds(j * GCHUNK, GCHUNK)]],
                    rows_v.at[pl.ds(j * GCHUNK, GCHUNK)],
                    sem,
                ).wait()
            pltpu.sync_copy(
                rows_v, out_hbm.at[pl.ds(b0, BW), pl.ds(f * DIM, DIM)]
            )
            return carry

        lax.fori_loop(0, N_FIELDS, field_body, 0)

    return fk(idx_t, flat).astype(jnp.float32)


# speedup vs baseline: 1.4027x; 1.4027x over previous
import functools

import jax
import jax.numpy as jnp
from jax import lax
from jax.experimental import pallas as pl
from jax.experimental.pallas import tpu as pltpu
from jax.experimental.pallas import tpu_sc as plsc

N_FIELDS = 26
VOCAB = 100000
DIM = 32
BATCH = 16384

VPAD = 100352
NVB = VPAD // 512
PACK_R = N_FIELDS * NVB * 128
G = 28

NC, NS = 2, 16
NW = NC * NS
BW = BATCH // NW
GCHUNK = 128
NG = BW // GCHUNK


def _tc_relayout(tbl_m):
    def body(x_ref, o_ref):
        xs = x_ref[...]
        for g in range(G):
            for q in range(4):
                o_ref[pl.ds(g * 128, 128), pl.ds(q * 32, 32)] = (
                    xs[:, (g * 4 + q) * 128:(g * 4 + q + 1) * 128].T
                )

    return pl.pallas_call(
        body,
        out_shape=jax.ShapeDtypeStruct((PACK_R, 128), jnp.float32),
        grid=(N_FIELDS, NVB // G),
        in_specs=[pl.BlockSpec((DIM, 512 * G), lambda f, v: (f, v))],
        out_specs=pl.BlockSpec(
            (128 * G, 128), lambda f, v: (f * (NVB // G) + v, 0)
        ),
    )(tbl_m)


def _sc_rowids(idx_t):
    mesh = plsc.VectorSubcoreMesh(core_axis_name="c", subcore_axis_name="s")

    @functools.partial(
        pl.kernel,
        out_type=jax.ShapeDtypeStruct((N_FIELDS * BATCH,), jnp.int32),
        mesh=mesh,
        compiler_params=pltpu.CompilerParams(use_tc_tiling_on_sc=True),
        scratch_types=[
            pltpu.VMEM((8, BW), jnp.int32),
        ],
    )
    def w1(idx_hbm, out_hbm, blk_v):
        wid = lax.axis_index("s") * NC + lax.axis_index("c")
        b0 = pl.multiple_of(wid * BW, BW)
        for fbase in (0, 8, 16, 24):
            nf = 8 if fbase < 24 else 2
            pltpu.sync_copy(
                idx_hbm.at[pl.ds(fbase, nf), pl.ds(b0, BW)],
                blk_v.at[pl.ds(0, nf)] if nf < 8 else blk_v,
            )
            for r in range(nf):
                f = fbase + r
                for k in range(BW // 16):
                    raw = blk_v[r, pl.ds(k * 16, 16)]
                    ok = (raw >= 0) & (raw < VOCAB)
                    vs = jnp.where(ok, raw + 1, 0)
                    blk_v[r, pl.ds(k * 16, 16)] = (
                        f * VPAD
                        + (vs >> 9) * 512
                        + (vs & 127) * 4
                        + ((vs >> 7) & 3)
                    )
                pltpu.sync_copy(
                    blk_v.at[r], out_hbm.at[pl.ds(f * BATCH + b0, BW)]
                )

    return w1(idx_t)


def kernel(indices, tables, weights):
    tbl_m = tables.transpose(0, 2, 1).reshape(N_FIELDS * DIM, VOCAB + 1)
    flat = _tc_relayout(tbl_m).reshape(PACK_R * 4, DIM)
    rowids = _sc_rowids(indices.T).reshape(N_FIELDS, BATCH)
    wb = jnp.broadcast_to(weights[:, None], (N_FIELDS, 16))
    mesh = plsc.VectorSubcoreMesh(core_axis_name="c", subcore_axis_name="s")

    @functools.partial(
        pl.kernel,
        out_type=jax.ShapeDtypeStruct((BATCH, N_FIELDS * DIM), jnp.float32),
        mesh=mesh,
        compiler_params=pltpu.CompilerParams(use_tc_tiling_on_sc=False),
        scratch_types=[
            pltpu.VMEM((N_FIELDS, 16), jnp.float32),
            pltpu.VMEM((BW,), jnp.int32),
            pltpu.VMEM((BW, DIM), jnp.float32),
            pltpu.SemaphoreType.DMA,
        ],
    )
    def fk(g_hbm, tbl_hbm, w_hbm, out_hbm, w_v, g_v, rows_v, sem):
        wid = lax.axis_index("s") * NC + lax.axis_index("c")
        b0 = pl.multiple_of(wid * BW, BW)
        pltpu.sync_copy(w_hbm, w_v)

        def field_body(f, carry):
            pltpu.sync_copy(g_hbm.at[f, pl.ds(b0, BW)], g_v)
            for j in range(NG):
                pltpu.async_copy(
                    tbl_hbm.at[g_v.at[pl.ds(j * GCHUNK, GCHUNK)]],
                    rows_v.at[pl.ds(j * GCHUNK, GCHUNK)],
                    sem,
                )
            for j in range(NG):
                pltpu.make_async_copy(
                    tbl_hbm.at[g_v.at[pl.ds(j * GCHUNK, GCHUNK)]],
                    rows_v.at[pl.ds(j * GCHUNK, GCHUNK)],
                    sem,
                ).wait()
            wvec = w_v[f]

            def mul_body(r, carry2):
                rows_v[r, pl.ds(0, 16)] = rows_v[r, pl.ds(0, 16)] * wvec
                rows_v[r, pl.ds(16, 16)] = rows_v[r, pl.ds(16, 16)] * wvec
                return carry2

            lax.fori_loop(0, BW, mul_body, 0, unroll=8)
            pltpu.sync_copy(
                rows_v, out_hbm.at[pl.ds(b0, BW), pl.ds(f * DIM, DIM)]
            )
            return carry

        lax.fori_loop(0, N_FIELDS, field_body, 0)

    return fk(rowids, flat, wb)


# speedup vs baseline: 1.4167x; 1.0099x over previous
import functools

import jax
import jax.numpy as jnp
from jax import lax
from jax.experimental import pallas as pl
from jax.experimental.pallas import tpu as pltpu
from jax.experimental.pallas import tpu_sc as plsc

N_FIELDS = 26
VOCAB = 100000
DIM = 32
BATCH = 16384

VPAD = 100352
NVB = VPAD // 512
PACK_R = N_FIELDS * NVB * 128
G = 49

NC, NS = 2, 16
NW = NC * NS
BW = BATCH // NW
GCHUNK = 128
NG = BW // GCHUNK


def _tc_relayout(tbl_m):
    def body(x_ref, o_ref):
        xs = x_ref[...]
        for g in range(G):
            for q in range(4):
                o_ref[pl.ds(g * 128, 128), pl.ds(q * 32, 32)] = (
                    xs[:, (g * 4 + q) * 128:(g * 4 + q + 1) * 128].T
                )

    return pl.pallas_call(
        body,
        out_shape=jax.ShapeDtypeStruct((PACK_R, 128), jnp.float32),
        grid=(N_FIELDS, NVB // G),
        in_specs=[pl.BlockSpec((DIM, 512 * G), lambda f, v: (f, v))],
        out_specs=pl.BlockSpec(
            (128 * G, 128), lambda f, v: (f * (NVB // G) + v, 0)
        ),
    )(tbl_m)


def _sc_rowids(idx_t):
    mesh = plsc.VectorSubcoreMesh(core_axis_name="c", subcore_axis_name="s")

    @functools.partial(
        pl.kernel,
        out_type=jax.ShapeDtypeStruct((N_FIELDS * BATCH,), jnp.int32),
        mesh=mesh,
        compiler_params=pltpu.CompilerParams(use_tc_tiling_on_sc=True),
        scratch_types=[
            pltpu.VMEM((8, BW), jnp.int32),
        ],
    )
    def w1(idx_hbm, out_hbm, blk_v):
        wid = lax.axis_index("s") * NC + lax.axis_index("c")
        b0 = pl.multiple_of(wid * BW, BW)
        for fbase in (0, 8, 16, 24):
            nf = 8 if fbase < 24 else 2
            pltpu.sync_copy(
                idx_hbm.at[pl.ds(fbase, nf), pl.ds(b0, BW)],
                blk_v.at[pl.ds(0, nf)] if nf < 8 else blk_v,
            )
            for r in range(nf):
                f = fbase + r
                for k in range(BW // 16):
                    raw = blk_v[r, pl.ds(k * 16, 16)]
                    ok = (raw >= 0) & (raw < VOCAB)
                    vs = jnp.where(ok, raw + 1, 0)
                    blk_v[r, pl.ds(k * 16, 16)] = (
                        f * VPAD
                        + (vs >> 9) * 512
                        + (vs & 127) * 4
                        + ((vs >> 7) & 3)
                    )
                pltpu.sync_copy(
                    blk_v.at[r], out_hbm.at[pl.ds(f * BATCH + b0, BW)]
                )

    return w1(idx_t)


def kernel(indices, tables, weights):
    tbl_m = tables.transpose(0, 2, 1).reshape(N_FIELDS * DIM, VOCAB + 1)
    flat = _tc_relayout(tbl_m).reshape(PACK_R * 4, DIM)
    rowids = _sc_rowids(indices.T).reshape(N_FIELDS, BATCH)
    wb = jnp.broadcast_to(weights[:, None], (N_FIELDS, 16))
    mesh = plsc.VectorSubcoreMesh(core_axis_name="c", subcore_axis_name="s")

    @functools.partial(
        pl.kernel,
        out_type=jax.ShapeDtypeStruct((BATCH, N_FIELDS * DIM), jnp.float32),
        mesh=mesh,
        compiler_params=pltpu.CompilerParams(use_tc_tiling_on_sc=False),
        scratch_types=[
            pltpu.VMEM((N_FIELDS, 16), jnp.float32),
            pltpu.VMEM((BW,), jnp.int32),
            pltpu.VMEM((BW, DIM), jnp.float32),
            pltpu.SemaphoreType.DMA,
        ],
    )
    def fk(g_hbm, tbl_hbm, w_hbm, out_hbm, w_v, g_v, rows_v, sem):
        wid = lax.axis_index("s") * NC + lax.axis_index("c")
        b0 = pl.multiple_of(wid * BW, BW)
        pltpu.sync_copy(w_hbm, w_v)

        def field_body(f, carry):
            pltpu.sync_copy(g_hbm.at[f, pl.ds(b0, BW)], g_v)
            for j in range(NG):
                pltpu.async_copy(
                    tbl_hbm.at[g_v.at[pl.ds(j * GCHUNK, GCHUNK)]],
                    rows_v.at[pl.ds(j * GCHUNK, GCHUNK)],
                    sem,
                )
            for j in range(NG):
                pltpu.make_async_copy(
                    tbl_hbm.at[g_v.at[pl.ds(j * GCHUNK, GCHUNK)]],
                    rows_v.at[pl.ds(j * GCHUNK, GCHUNK)],
                    sem,
                ).wait()
            wvec = w_v[f]

            def mul_body(r, carry2):
                rows_v[r, pl.ds(0, 16)] = rows_v[r, pl.ds(0, 16)] * wvec
                rows_v[r, pl.ds(16, 16)] = rows_v[r, pl.ds(16, 16)] * wvec
                return carry2

            lax.fori_loop(0, BW, mul_body, 0, unroll=8)
            pltpu.sync_copy(
                rows_v, out_hbm.at[pl.ds(b0, BW), pl.ds(f * DIM, DIM)]
            )
            return carry

        lax.fori_loop(0, N_FIELDS, field_body, 0)

    return fk(rowids, flat, wb)


# speedup vs baseline: 1.4238x; 1.0050x over previous
import functools

import jax
import jax.numpy as jnp
from jax import lax
from jax.experimental import pallas as pl
from jax.experimental.pallas import tpu as pltpu
from jax.experimental.pallas import tpu_sc as plsc

N_FIELDS = 26
VOCAB = 100000
DIM = 32
BATCH = 16384

VPAD = 100352
NVB = VPAD // 512
PACK_R = N_FIELDS * NVB * 128
G = 98

NC, NS = 2, 16
NW = NC * NS
BW = BATCH // NW
GCHUNK = 128
NG = BW // GCHUNK


def _tc_relayout(tbl_m):
    def body(x_ref, o_ref):
        xs = x_ref[...]
        for g in range(G):
            for q in range(4):
                o_ref[pl.ds(g * 128, 128), pl.ds(q * 32, 32)] = (
                    xs[:, (g * 4 + q) * 128:(g * 4 + q + 1) * 128].T
                )

    return pl.pallas_call(
        body,
        out_shape=jax.ShapeDtypeStruct((PACK_R, 128), jnp.float32),
        grid=(N_FIELDS, NVB // G),
        in_specs=[pl.BlockSpec((DIM, 512 * G), lambda f, v: (f, v))],
        out_specs=pl.BlockSpec(
            (128 * G, 128), lambda f, v: (f * (NVB // G) + v, 0)
        ),
    )(tbl_m)


def _sc_rowids(idx_t):
    mesh = plsc.VectorSubcoreMesh(core_axis_name="c", subcore_axis_name="s")

    @functools.partial(
        pl.kernel,
        out_type=jax.ShapeDtypeStruct((N_FIELDS * BATCH,), jnp.int32),
        mesh=mesh,
        compiler_params=pltpu.CompilerParams(use_tc_tiling_on_sc=True),
        scratch_types=[
            pltpu.VMEM((8, BW), jnp.int32),
        ],
    )
    def w1(idx_hbm, out_hbm, blk_v):
        wid = lax.axis_index("s") * NC + lax.axis_index("c")
        b0 = pl.multiple_of(wid * BW, BW)
        for fbase in (0, 8, 16, 24):
            nf = 8 if fbase < 24 else 2
            pltpu.sync_copy(
                idx_hbm.at[pl.ds(fbase, nf), pl.ds(b0, BW)],
                blk_v.at[pl.ds(0, nf)] if nf < 8 else blk_v,
            )
            for r in range(nf):
                f = fbase + r
                for k in range(BW // 16):
                    raw = blk_v[r, pl.ds(k * 16, 16)]
                    ok = (raw >= 0) & (raw < VOCAB)
                    vs = jnp.where(ok, raw + 1, 0)
                    blk_v[r, pl.ds(k * 16, 16)] = (
                        f * VPAD
                        + (vs >> 9) * 512
                        + (vs & 127) * 4
                        + ((vs >> 7) & 3)
                    )
                pltpu.sync_copy(
                    blk_v.at[r], out_hbm.at[pl.ds(f * BATCH + b0, BW)]
                )

    return w1(idx_t)


def kernel(indices, tables, weights):
    tbl_m = tables.transpose(0, 2, 1).reshape(N_FIELDS * DIM, VOCAB + 1)
    flat = _tc_relayout(tbl_m).reshape(PACK_R * 4, DIM)
    rowids = _sc_rowids(indices.T).reshape(N_FIELDS, BATCH)
    wb = jnp.broadcast_to(weights[:, None], (N_FIELDS, 16))
    mesh = plsc.VectorSubcoreMesh(core_axis_name="c", subcore_axis_name="s")

    @functools.partial(
        pl.kernel,
        out_type=jax.ShapeDtypeStruct((BATCH, N_FIELDS * DIM), jnp.float32),
        mesh=mesh,
        compiler_params=pltpu.CompilerParams(use_tc_tiling_on_sc=False),
        scratch_types=[
            pltpu.VMEM((N_FIELDS, 16), jnp.float32),
            pltpu.VMEM((BW,), jnp.int32),
            pltpu.VMEM((BW, DIM), jnp.float32),
            pltpu.SemaphoreType.DMA,
        ],
    )
    def fk(g_hbm, tbl_hbm, w_hbm, out_hbm, w_v, g_v, rows_v, sem):
        wid = lax.axis_index("s") * NC + lax.axis_index("c")
        b0 = pl.multiple_of(wid * BW, BW)
        pltpu.sync_copy(w_hbm, w_v)

        def field_body(f, carry):
            pltpu.sync_copy(g_hbm.at[f, pl.ds(b0, BW)], g_v)
            for j in range(NG):
                pltpu.async_copy(
                    tbl_hbm.at[g_v.at[pl.ds(j * GCHUNK, GCHUNK)]],
                    rows_v.at[pl.ds(j * GCHUNK, GCHUNK)],
                    sem,
                )
            for j in range(NG):
                pltpu.make_async_copy(
                    tbl_hbm.at[g_v.at[pl.ds(j * GCHUNK, GCHUNK)]],
                    rows_v.at[pl.ds(j * GCHUNK, GCHUNK)],
                    sem,
                ).wait()
            wvec = w_v[f]

            def mul_body(r, carry2):
                rows_v[r, pl.ds(0, 16)] = rows_v[r, pl.ds(0, 16)] * wvec
                rows_v[r, pl.ds(16, 16)] = rows_v[r, pl.ds(16, 16)] * wvec
                return carry2

            lax.fori_loop(0, BW, mul_body, 0, unroll=8)
            pltpu.sync_copy(
                rows_v, out_hbm.at[pl.ds(b0, BW), pl.ds(f * DIM, DIM)]
            )
            return carry

        lax.fori_loop(0, N_FIELDS, field_body, 0)

    return fk(rowids, flat, wb)
